# baseline (speedup 1.0000x reference)
import jax
import jax.numpy as jnp
from jax import lax
from jax.experimental import pallas as pl
from jax.experimental.pallas import tpu as pltpu
from jax.experimental.pallas import tpu_sc as plsc

B = 1024
S = 200
E = 64
N = B * S
EPS = 1e-3

NC = 2
NS = 16
NW = NC * NS
ROWS_PER_TILE = N // NW
CHUNK = 400
NCHUNK = ROWS_PER_TILE // CHUNK
SUBS = ((0, 128), (128, 128), (256, 128), (384, 16))

_MAGIC = 0x5F3759DF


def _body(ids_hbm, table_hbm, pos_hbm, gamma_hbm, beta_hbm, out_hbm,
          pos_v, g_v, b_v, idx0, idx1, rows0, rows1, outb0, outb1,
          gsem0, gsem1, ssem0, ssem1):
    idx = (idx0, idx1)
    rows = (rows0, rows1)
    outb = (outb0, outb1)
    gsem = (gsem0, gsem1)
    ssem = (ssem0, ssem1)
    wid = lax.axis_index("s") * NC + lax.axis_index("c")
    row0 = wid * ROWS_PER_TILE

    pltpu.sync_copy(pos_hbm.at[pl.ds(0, S)], pos_v)
    pltpu.sync_copy(gamma_hbm, g_v)
    pltpu.sync_copy(beta_hbm, b_v)
    g = [g_v[pl.ds(i * 16, 16)] for i in range(4)]
    bta = [b_v[pl.ds(i * 16, 16)] for i in range(4)]

    def stage(k):
        buf = k % 2
        base = row0 + k * CHUNK
        pltpu.sync_copy(ids_hbm.at[pl.ds(base, CHUNK)], idx[buf])
        handles = []
        for off, ln in SUBS:
            handles.append(pltpu.async_copy(
                table_hbm.at[idx[buf].at[pl.ds(off, ln)]],
                rows[buf].at[pl.ds(off, ln)],
                gsem[buf]))
        return handles

    def compute(buf):
        rv = rows[buf]
        ov = outb[buf]

        def body(s, c):
            p = [pos_v[s, pl.ds(i * 16, 16)] for i in range(4)]
            for seq in range(CHUNK // S):
                r = seq * S + s
                t = [rv[r, pl.ds(i * 16, 16)] + p[i] for i in range(4)]
                sv = (t[0] + t[1]) + (t[2] + t[3])
                sq = (t[0] * t[0] + t[1] * t[1]) + (t[2] * t[2] + t[3] * t[3])
                tot = jnp.broadcast_to(jnp.sum(sv), (16,))
                tot2 = jnp.broadcast_to(jnp.sum(sq), (16,))
                mean = tot * (1.0 / E)
                var = tot2 * (1.0 / E) - mean * mean
                x = var + EPS
                iv = jnp.int32(_MAGIC) - lax.shift_right_logical(
                    lax.bitcast_convert_type(x, jnp.int32), 1)
                y = lax.bitcast_convert_type(iv, jnp.float32)
                y = y * (1.5 - 0.5 * x * y * y)
                y = y * (1.5 - 0.5 * x * y * y)
                for i in range(4):
                    a = y * g[i]
                    c0 = bta[i] - mean * a
                    ov[r, pl.ds(i * 16, 16)] = t[i] * a + c0
            return c
        lax.fori_loop(0, S, body, 0)

    pend_g = {0: stage(0)}
    pend_s = {}
    for k in range(NCHUNK):
        buf = k % 2
        if k + 1 < NCHUNK:
            pend_g[k + 1] = stage(k + 1)
        for h in pend_g.pop(k):
            h.wait()
        if k - 2 in pend_s:
            pend_s.pop(k - 2).wait()
        compute(buf)
        pend_s[k] = pltpu.async_copy(
            outb[buf], out_hbm.at[pl.ds(row0 + k * CHUNK, CHUNK)],
            ssem[buf])
    for k in sorted(pend_s):
        pend_s.pop(k).wait()


@jax.jit
def kernel(input_ids, word_table, pos_table, gamma, beta):
    ids_flat = input_ids.reshape(N).astype(jnp.int32)
    mesh = plsc.VectorSubcoreMesh(core_axis_name="c", subcore_axis_name="s")
    run = pl.kernel(
        _body,
        out_type=jax.ShapeDtypeStruct((N, E), jnp.float32),
        mesh=mesh,
        scratch_types=[
            pltpu.VMEM((S, E), jnp.float32),
            pltpu.VMEM((E,), jnp.float32),
            pltpu.VMEM((E,), jnp.float32),
            pltpu.VMEM((CHUNK,), jnp.int32),
            pltpu.VMEM((CHUNK,), jnp.int32),
            pltpu.VMEM((CHUNK, E), jnp.float32),
            pltpu.VMEM((CHUNK, E), jnp.float32),
            pltpu.VMEM((CHUNK, E), jnp.float32),
            pltpu.VMEM((CHUNK, E), jnp.float32),
            pltpu.SemaphoreType.DMA,
            pltpu.SemaphoreType.DMA,
            pltpu.SemaphoreType.DMA,
            pltpu.SemaphoreType.DMA,
        ],
        compiler_params=pltpu.CompilerParams(
            needs_layout_passes=False, use_tc_tiling_on_sc=False),
    )
    out = run(ids_flat, word_table, pos_table, gamma, beta)
    return out.reshape(B, S, E)


# speedup vs baseline: 1.0017x; 1.0017x over previous
import jax
import jax.numpy as jnp
from jax import lax
from jax.experimental import pallas as pl
from jax.experimental.pallas import tpu as pltpu
from jax.experimental.pallas import tpu_sc as plsc

B = 1024
S = 200
E = 64
N = B * S
EPS = 1e-3

NC = 2
NS = 16
NW = NC * NS
ROWS_PER_TILE = N // NW
CHUNK = 400
NCHUNK = ROWS_PER_TILE // CHUNK
SUBS = ((0, 128), (128, 128), (256, 128), (384, 16))

_MAGIC = 0x5F3759DF


def _body(ids_hbm, table_hbm, pos_hbm, gamma_hbm, beta_hbm, out_hbm,
          pos_v, g_v, b_v, idx0, idx1, rows0, rows1, outb0, outb1,
          gsem0, gsem1, ssem0, ssem1):
    idx = (idx0, idx1)
    rows = (rows0, rows1)
    outb = (outb0, outb1)
    gsem = (gsem0, gsem1)
    ssem = (ssem0, ssem1)
    wid = lax.axis_index("s") * NC + lax.axis_index("c")
    row0 = wid * ROWS_PER_TILE

    pltpu.sync_copy(pos_hbm.at[pl.ds(0, S)], pos_v)
    pltpu.sync_copy(gamma_hbm, g_v)
    pltpu.sync_copy(beta_hbm, b_v)
    g = [g_v[pl.ds(i * 16, 16)] for i in range(4)]
    bta = [b_v[pl.ds(i * 16, 16)] for i in range(4)]

    def stage(k):
        buf = k % 2
        base = row0 + k * CHUNK
        pltpu.sync_copy(ids_hbm.at[pl.ds(base, CHUNK)], idx[buf])
        handles = []
        for off, ln in SUBS:
            handles.append(pltpu.async_copy(
                table_hbm.at[idx[buf].at[pl.ds(off, ln)]],
                rows[buf].at[pl.ds(off, ln)],
                gsem[buf]))
        return handles

    def compute(buf):
        rv = rows[buf]
        ov = outb[buf]

        def body(s, c):
            p = [pos_v[s, pl.ds(i * 16, 16)] for i in range(4)]
            for seq in range(CHUNK // S):
                r = seq * S + s
                t = [rv[r, pl.ds(i * 16, 16)] + p[i] for i in range(4)]
                sv = (t[0] + t[1]) + (t[2] + t[3])
                sq = (t[0] * t[0] + t[1] * t[1]) + (t[2] * t[2] + t[3] * t[3])
                tot = jnp.broadcast_to(jnp.sum(sv), (16,))
                tot2 = jnp.broadcast_to(jnp.sum(sq), (16,))
                mean = tot * (1.0 / E)
                var = tot2 * (1.0 / E) - mean * mean
                x = var + EPS
                iv = jnp.int32(_MAGIC) - lax.shift_right_logical(
                    lax.bitcast_convert_type(x, jnp.int32), 1)
                y0 = lax.bitcast_convert_type(iv, jnp.float32)
                h = 0.5 * x
                y = y0 * (1.5 - h * y0 * y0)
                y = y * (1.5 - h * y * y)
                for i in range(4):
                    a = y * g[i]
                    c0 = bta[i] - mean * a
                    ov[r, pl.ds(i * 16, 16)] = t[i] * a + c0
            return c
        lax.fori_loop(0, S, body, 0)

    pend_g = {0: stage(0)}
    pend_s = {}
    for k in range(NCHUNK):
        buf = k % 2
        if k + 1 < NCHUNK:
            pend_g[k + 1] = stage(k + 1)
        for h in pend_g.pop(k):
            h.wait()
        if k - 2 in pend_s:
            pend_s.pop(k - 2).wait()
        compute(buf)
        pend_s[k] = pltpu.async_copy(
            outb[buf], out_hbm.at[pl.ds(row0 + k * CHUNK, CHUNK)],
            ssem[buf])
    for k in sorted(pend_s):
        pend_s.pop(k).wait()


@jax.jit
def kernel(input_ids, word_table, pos_table, gamma, beta):
    ids_flat = input_ids.reshape(N).astype(jnp.int32)
    mesh = plsc.VectorSubcoreMesh(core_axis_name="c", subcore_axis_name="s")
    run = pl.kernel(
        _body,
        out_type=jax.ShapeDtypeStruct((N, E), jnp.float32),
        mesh=mesh,
        scratch_types=[
            pltpu.VMEM((S, E), jnp.float32),
            pltpu.VMEM((E,), jnp.float32),
            pltpu.VMEM((E,), jnp.float32),
            pltpu.VMEM((CHUNK,), jnp.int32),
            pltpu.VMEM((CHUNK,), jnp.int32),
            pltpu.VMEM((CHUNK, E), jnp.float32),
            pltpu.VMEM((CHUNK, E), jnp.float32),
            pltpu.VMEM((CHUNK, E), jnp.float32),
            pltpu.VMEM((CHUNK, E), jnp.float32),
            pltpu.SemaphoreType.DMA,
            pltpu.SemaphoreType.DMA,
            pltpu.SemaphoreType.DMA,
            pltpu.SemaphoreType.DMA,
        ],
        compiler_params=pltpu.CompilerParams(
            needs_layout_passes=False, use_tc_tiling_on_sc=False),
    )
    out = run(ids_flat, word_table, pos_table, gamma, beta)
    return out.reshape(B, S, E)
